# baseline (speedup 1.0000x reference)
import functools

import jax
import jax.numpy as jnp
from jax import lax
from jax.experimental import pallas as pl
from jax.experimental.pallas import tpu as pltpu
from jax.experimental.pallas import tpu_sc as plsc

_NUM_CORES = 2
_NUM_SUBCORES = 16
_LANES = 16


def _sc_body(B, N, F, H, W, row_hbm, col_hbm, out_hbm, col_v, row_v, chunk_v,
             sem):
    HW = H * W
    n_workers = _NUM_CORES * _NUM_SUBCORES
    CH = 2 * F // n_workers
    wid = lax.axis_index("s") * _NUM_CORES + lax.axis_index("c")
    c0 = wid * CH

    pltpu.sync_copy(col_hbm, col_v)
    pltpu.sync_copy(row_hbm, row_v)

    is_x = c0 < F
    cbase = jnp.where(is_x, c0, c0 - F)
    lane = lax.iota(jnp.int32, _LANES)
    n_grp = HW // _LANES
    xidx = [(j * _LANES) % W + lane for j in range(n_grp)]
    yidx = [
        jnp.full((_LANES,), (j * _LANES) // W, jnp.int32)
        for j in range(n_grp)
    ]

    for i in range(CH):
        cvec = jnp.full((_LANES,), cbase + i, jnp.int32)
        for j in range(n_grp):
            gx = plsc.load_gather(col_v, [xidx[j], cvec])
            gy = plsc.load_gather(row_v, [yidx[j], cvec])
            chunk_v[i, pl.ds(j * _LANES, _LANES)] = jnp.where(is_x, gx, gy)

    copies = [
        pltpu.async_copy(chunk_v, out_hbm.at[b, pl.ds(c0, CH)], sem)
        for b in range(B)
    ]
    for cp in copies:
        cp.wait()


def kernel(input, row_embed, col_embed):
    B, C, H, W = input.shape
    N, F = row_embed.shape
    n_workers = _NUM_CORES * _NUM_SUBCORES
    CH = 2 * F // n_workers
    mesh = plsc.VectorSubcoreMesh(core_axis_name="c", subcore_axis_name="s")
    k = functools.partial(
        pl.kernel,
        out_type=jax.ShapeDtypeStruct((B, 2 * F, H * W), row_embed.dtype),
        mesh=mesh,
        scratch_types=[
            pltpu.VMEM((N, F), jnp.float32),
            pltpu.VMEM((N, F), jnp.float32),
            pltpu.VMEM((CH, H * W), jnp.float32),
            pltpu.SemaphoreType.DMA,
        ],
        compiler_params=pltpu.CompilerParams(needs_layout_passes=False),
    )(functools.partial(_sc_body, B, N, F, H, W))
    out = k(row_embed, col_embed)
    return out.reshape(B, 2 * F, H, W)


# speedup vs baseline: 2.0072x; 2.0072x over previous
import functools

import jax
import jax.numpy as jnp
from jax import lax
from jax.experimental import pallas as pl
from jax.experimental.pallas import tpu as pltpu

_NSEM = 8


def _pos_body(B, H, W, row_ref, col_ref, out_ref, scratch, sems):
    F = row_ref.shape[1]
    HW = H * W
    lane_w = lax.broadcasted_iota(jnp.int32, (W, HW), 1)
    sub_w = lax.broadcasted_iota(jnp.int32, (W, HW), 0)
    tile_sel = (lane_w % W == sub_w).astype(jnp.float32)
    lane_h = lax.broadcasted_iota(jnp.int32, (H, HW), 1)
    sub_h = lax.broadcasted_iota(jnp.int32, (H, HW), 0)
    rep_sel = (lane_h // W == sub_h).astype(jnp.float32)
    dn = (((0,), (0,)), ((), ()))
    scratch[:F] = lax.dot_general(
        col_ref[:W, :], tile_sel, dn, preferred_element_type=jnp.float32)
    scratch[F:] = lax.dot_general(
        row_ref[:H, :], rep_sel, dn, preferred_element_type=jnp.float32)
    for b in range(B):
        pltpu.make_async_copy(
            scratch, out_ref.at[b], sems.at[b % _NSEM]).start()
    for b in range(B):
        pltpu.make_async_copy(
            scratch, out_ref.at[b], sems.at[b % _NSEM]).wait()


def kernel(input, row_embed, col_embed):
    B, C, H, W = input.shape
    N, F = row_embed.shape
    out = pl.pallas_call(
        functools.partial(_pos_body, B, H, W),
        in_specs=[
            pl.BlockSpec(memory_space=pltpu.MemorySpace.VMEM),
            pl.BlockSpec(memory_space=pltpu.MemorySpace.VMEM),
        ],
        out_specs=pl.BlockSpec(memory_space=pltpu.MemorySpace.HBM),
        out_shape=jax.ShapeDtypeStruct((B, 2 * F, H * W), row_embed.dtype),
        scratch_shapes=[
            pltpu.VMEM((2 * F, H * W), jnp.float32),
            pltpu.SemaphoreType.DMA((_NSEM,)),
        ],
    )(row_embed, col_embed)
    return out.reshape(B, 2 * F, H, W)
